# Initial kernel scaffold; baseline (speedup 1.0000x reference)
import jax
import jax.numpy as jnp
from jax.experimental import pallas as pl


def kernel(x, gate_w, w1, v1, w2):
    raise NotImplementedError("write your pallas kernel here")


# speedup vs baseline: 1.1722x; 1.1722x over previous
import jax
import jax.numpy as jnp
from jax.experimental import pallas as pl

D_MODEL = 2048
FFN = 2048
E = 8
T = 2048
BF = 256


def _router_kernel(x_ref, gw_ref, dw_ref):
    logits = jax.lax.dot_general(
        x_ref[...], gw_ref[...], (((1,), (1,)), ((), ())),
        preferred_element_type=jnp.float32)
    m = jnp.max(logits, axis=1, keepdims=True)
    p = jnp.exp(logits - m)
    p = p / jnp.sum(p, axis=1, keepdims=True)
    lane = jax.lax.broadcasted_iota(jnp.int32, p.shape, 1)
    m1 = jnp.max(p, axis=1, keepdims=True)
    i1 = jnp.min(jnp.where(p == m1, lane, E), axis=1, keepdims=True)
    p2 = jnp.where(lane == i1, -1.0, p)
    m2 = jnp.max(p2, axis=1, keepdims=True)
    i2 = jnp.min(jnp.where(p2 == m2, lane, E), axis=1, keepdims=True)
    s = m1 + m2
    dw_ref[...] = (jnp.where(lane == i1, m1 / s, 0.0)
                   + jnp.where(lane == i2, m2 / s, 0.0))


def _moe_kernel(x_ref, dw_ref, w1_ref, v1_ref, w2_ref, out_ref):
    e = pl.program_id(0)
    f = pl.program_id(1)

    @pl.when((e == 0) & (f == 0))
    def _():
        out_ref[...] = jnp.zeros_like(out_ref)

    xb = x_ref[...]
    w1t = w1_ref[0]
    v1t = v1_ref[0]
    w2t = w2_ref[0]
    a = jax.lax.dot_general(xb, w1t, (((1,), (1,)), ((), ())),
                            preferred_element_type=jnp.float32)
    b = jax.lax.dot_general(xb, v1t, (((1,), (1,)), ((), ())),
                            preferred_element_type=jnp.float32)
    h = (a * jax.nn.sigmoid(a) * b).astype(jnp.bfloat16)
    y = jax.lax.dot_general(h, w2t, (((1,), (0,)), ((), ())),
                            preferred_element_type=jnp.float32)
    lane_e = jax.lax.broadcasted_iota(jnp.int32, dw_ref.shape, 1)
    col = jnp.sum(jnp.where(lane_e == e, dw_ref[...], 0.0), axis=1,
                  keepdims=True)
    out_ref[...] += col * y


def kernel(x, gate_w, w1, v1, w2):
    dense_w = pl.pallas_call(
        _router_kernel,
        out_shape=jax.ShapeDtypeStruct((T, E), jnp.float32),
    )(x, gate_w)

    x_bf = x.astype(jnp.bfloat16)
    w1r = w1.astype(jnp.bfloat16).reshape(E, FFN, D_MODEL)
    v1r = v1.astype(jnp.bfloat16).reshape(E, FFN, D_MODEL)
    w2r = w2.astype(jnp.bfloat16).reshape(E, FFN, D_MODEL)

    grid = (E, FFN // BF)
    out = pl.pallas_call(
        _moe_kernel,
        grid=grid,
        in_specs=[
            pl.BlockSpec((T, D_MODEL), lambda e, f: (0, 0)),
            pl.BlockSpec((T, E), lambda e, f: (0, 0)),
            pl.BlockSpec((1, BF, D_MODEL), lambda e, f: (e, f, 0)),
            pl.BlockSpec((1, BF, D_MODEL), lambda e, f: (e, f, 0)),
            pl.BlockSpec((1, BF, D_MODEL), lambda e, f: (e, f, 0)),
        ],
        out_specs=pl.BlockSpec((T, D_MODEL), lambda e, f: (0, 0)),
        out_shape=jax.ShapeDtypeStruct((T, D_MODEL), jnp.float32),
    )(x_bf, dense_w, w1r, v1r, w2r)
    return out
